# baseline (speedup 1.0000x reference)
import functools

import jax
import jax.numpy as jnp
from jax import lax
from jax.experimental import pallas as pl
from jax.experimental.pallas import tpu as pltpu
from jax.experimental.pallas import tpu_sc as plsc

NC = 2
NS = 16
NW = NC * NS
K = 4


@functools.partial(jax.jit, static_argnums=(2,))
def _emb_lookup(x, table, dim):
    batch, hist = x.shape
    rows_w = batch // NW
    chunks = [(off, min(128, hist - off)) for off in range(0, hist, 128)]

    @functools.partial(
        pl.kernel,
        mesh=plsc.VectorSubcoreMesh(core_axis_name="c", subcore_axis_name="s"),
        out_type=jax.ShapeDtypeStruct((batch, hist, dim), jnp.float32),
        scratch_types=[
            pltpu.VMEM((rows_w, hist), jnp.int32),
            pltpu.VMEM((K, hist, dim), jnp.float32),
            pltpu.SemaphoreType.DMA((K,)),
        ],
        compiler_params=pltpu.CompilerParams(use_tc_tiling_on_sc=False),
    )
    def body(x_hbm, table_hbm, out_hbm, idx_v, rows_v, gsem):
        wid = lax.axis_index("s") * NC + lax.axis_index("c")
        row0 = wid * rows_w
        pltpu.sync_copy(x_hbm.at[pl.ds(row0, rows_w)], idx_v)

        def start(r, b):
            for off, cnt in chunks:
                pltpu.async_copy(
                    table_hbm.at[idx_v.at[r, pl.ds(off, cnt)]],
                    rows_v.at[b, pl.ds(off, cnt)],
                    gsem.at[b],
                )

        def wait(r, b):
            for off, cnt in chunks:
                pltpu.make_async_copy(
                    table_hbm.at[idx_v.at[r, pl.ds(off, cnt)]],
                    rows_v.at[b, pl.ds(off, cnt)],
                    gsem.at[b],
                ).wait()

        for b in range(K):
            start(b, b)

        def loop(g, carry):
            for b in range(K):
                r = g * K + b
                wait(r, b)
                pltpu.sync_copy(rows_v.at[b], out_hbm.at[row0 + r])
                nr = r + K

                @pl.when(nr < rows_w)
                def _():
                    start(nr, b)

            return carry

        lax.fori_loop(0, rows_w // K, loop, 0)

    return body(x, table)


def kernel(x, table):
    b, h = x.shape
    v, d = table.shape
    assert b % (NW * K) == 0
    return _emb_lookup(x.astype(jnp.int32), table, d)


# speedup vs baseline: 1.3348x; 1.3348x over previous
import functools

import jax
import jax.numpy as jnp
from jax import lax
from jax.experimental import pallas as pl
from jax.experimental.pallas import tpu as pltpu
from jax.experimental.pallas import tpu_sc as plsc

NC = 2
NS = 16
NW = NC * NS
CH = 128
K = 8


@functools.partial(jax.jit, static_argnums=(2,))
def _emb_lookup(x_blk, table, dim):
    n = x_blk.shape[0] * x_blk.shape[1]
    per_w = n // NW
    nchunk = per_w // CH
    rows_w = per_w // 128

    @functools.partial(
        pl.kernel,
        mesh=plsc.VectorSubcoreMesh(core_axis_name="c", subcore_axis_name="s"),
        out_type=jax.ShapeDtypeStruct((n, 128), jnp.float32),
        scratch_types=[
            pltpu.VMEM((rows_w, 128), jnp.int32),
            pltpu.VMEM((K, CH, 64), jnp.float32),
            pltpu.SemaphoreType.DMA((K,)),
        ],
        compiler_params=pltpu.CompilerParams(use_tc_tiling_on_sc=False),
    )
    def body(x_hbm, table_hbm, out_hbm, idx_v, rows_v, gsem):
        wid = lax.axis_index("s") * NC + lax.axis_index("c")
        base = wid * per_w
        pltpu.sync_copy(x_hbm.at[pl.ds(wid * rows_w, rows_w)], idx_v)

        def start(j, b):
            pltpu.async_copy(table_hbm.at[idx_v.at[j]], rows_v.at[b], gsem.at[b])

        def wait(j, b):
            pltpu.make_async_copy(
                table_hbm.at[idx_v.at[j]], rows_v.at[b], gsem.at[b]
            ).wait()

        for b in range(K):
            start(b, b)

        def loop(g, carry):
            for b in range(K):
                j = g * K + b
                wait(j, b)
                pltpu.sync_copy(
                    rows_v.at[b],
                    out_hbm.at[pl.ds(base + j * CH, CH), pl.ds(0, 64)],
                )
                nj = j + K

                @pl.when(nj < nchunk)
                def _():
                    start(nj, b)

            return carry

        lax.fori_loop(0, nchunk // K, loop, 0)

    return body(x_blk, table)


def kernel(x, table):
    b, h = x.shape
    v, d = table.shape
    n = b * h
    assert d == 64 and n % (NW * CH * K) == 0
    x_blk = x.reshape(-1, 128).astype(jnp.int32)
    out2 = _emb_lookup(x_blk, table, d)
    return out2[:, :d].reshape(b, h, d)
